# baseline (speedup 1.0000x reference)
import functools

import jax
import jax.numpy as jnp
from jax import lax
from jax.experimental import pallas as pl
from jax.experimental.pallas import tpu as pltpu
from jax.experimental.pallas import tpu_sc as plsc

_DIM = 32
_CODES = 512
_BM = 2048


def _argmin_body(n_tokens, x_ref, emb_ref, idx_ref, loss_ref, tab_ref):
    i = pl.program_id(0)
    x = x_ref[...]
    e = emb_ref[...]
    rowsq = jnp.sum(x * x, axis=1, keepdims=True)
    esq = jnp.sum(e * e, axis=0, keepdims=True)
    dot = jnp.dot(x, e, preferred_element_type=jnp.float32)
    dist = (rowsq - 2.0 * dot) + esq
    minval = jnp.min(dist, axis=1, keepdims=True)
    cols = lax.broadcasted_iota(jnp.int32, (1, _CODES), 1).astype(jnp.float32)
    idxf = jnp.min(jnp.where(dist == minval, cols, float(_CODES)), axis=1)
    idx_ref[...] = idxf.astype(jnp.int32).reshape(_BM // 256, 256)
    part = jnp.dot(jnp.ones((1, _BM), jnp.float32), minval,
                   preferred_element_type=jnp.float32)

    @pl.when(i == 0)
    def _init():
        loss_ref[...] = jnp.zeros((1, 1), jnp.float32)
        tab_ref[...] = e.T

    loss_ref[...] += part

    @pl.when(i == pl.num_programs(0) - 1)
    def _scale():
        loss_ref[...] *= 1.25 / (n_tokens * _DIM)


def _tc_argmin(flat_x, embeddings, chunk, n_chunks):
    n_total = flat_x.shape[0]
    nc = n_total // n_chunks
    grid = nc // _BM
    base = chunk * grid
    return pl.pallas_call(
        functools.partial(_argmin_body, n_total),
        grid=(grid,),
        in_specs=[
            pl.BlockSpec((_BM, _DIM), lambda i: (base + i, 0)),
            pl.BlockSpec((_DIM, _CODES), lambda i: (0, 0)),
        ],
        out_specs=[
            pl.BlockSpec((_BM // 256, 256), lambda i: (i, 0)),
            pl.BlockSpec((1, 1), lambda i: (0, 0)),
            pl.BlockSpec((_CODES, _DIM), lambda i: (0, 0)),
        ],
        out_shape=[
            jax.ShapeDtypeStruct((nc // 256, 256), jnp.int32),
            jax.ShapeDtypeStruct((1, 1), jnp.float32),
            jax.ShapeDtypeStruct((_CODES, _DIM), jnp.float32),
        ],
    )(flat_x, embeddings)


_SC_CORES = 2
_SC_SUBCORES = 16


def _make_sc_gather(n_tokens):
    nw = _SC_CORES * _SC_SUBCORES
    rows_per_w = n_tokens // nw
    fire = 256
    group = 4
    gtok = group * fire
    n_groups = rows_per_w // gtok
    n_idx_rows = rows_per_w // fire
    d0 = n_tokens // gtok
    mesh = plsc.VectorSubcoreMesh(
        core_axis_name="c", subcore_axis_name="s",
        num_cores=_SC_CORES, num_subcores=_SC_SUBCORES)

    @functools.partial(
        pl.kernel,
        mesh=mesh,
        out_type=jax.ShapeDtypeStruct((d0, gtok, _DIM), jnp.float32),
        scratch_types=[
            pltpu.VMEM((n_idx_rows, fire), jnp.int32),
            pltpu.VMEM((2, gtok, _DIM), jnp.float32),
            pltpu.SemaphoreType.DMA,
            pltpu.SemaphoreType.DMA,
            pltpu.SemaphoreType.DMA,
            pltpu.SemaphoreType.DMA,
        ],
        compiler_params=pltpu.CompilerParams(use_tc_tiling_on_sc=False),
    )
    def gather_kernel(table_hbm, idx_hbm, out_hbm, idx_v, rows_v,
                      gsem0, gsem1, ssem0, ssem1):
        wid = lax.axis_index("s") * _SC_CORES + lax.axis_index("c")
        gsems = (gsem0, gsem1)
        ssems = (ssem0, ssem1)
        pltpu.sync_copy(idx_hbm.at[pl.ds(wid * n_idx_rows, n_idx_rows)], idx_v)

        def fire_gathers(g, b):
            return [pltpu.async_copy(
                table_hbm.at[idx_v.at[g * group + t]],
                rows_v.at[b, pl.ds(t * fire, fire)],
                gsems[b]) for t in range(group)]

        gh = {0: fire_gathers(0, 0)}
        sh = {}
        for g in range(n_groups):
            b = g & 1
            nb = b ^ 1
            if g + 1 < n_groups:
                if g - 1 in sh:
                    sh.pop(g - 1).wait()
                gh[g + 1] = fire_gathers(g + 1, nb)
            for c in gh.pop(g):
                c.wait()
            sh[g] = pltpu.async_copy(
                rows_v.at[b], out_hbm.at[wid * n_groups + g], ssems[b])
        for g in sorted(sh):
            sh.pop(g).wait()

    return gather_kernel


def kernel(inputs, embeddings):
    in_shape = inputs.shape
    flat_x = inputs.reshape(-1, _DIM)
    n = flat_x.shape[0]
    n_chunks = 2
    gather = _make_sc_gather(n // n_chunks)
    parts, losses, table = [], [], None
    for c in range(n_chunks):
        idx_c, loss_c, tab_c = _tc_argmin(flat_x, embeddings, c, n_chunks)
        losses.append(loss_c)
        if table is None:
            table = tab_c
        parts.append(gather(table, idx_c))
    quant = jnp.concatenate(parts, axis=0)
    loss = sum(l[0, 0] for l in losses)
    return quant.reshape(in_shape), loss


# speedup vs baseline: 1.0158x; 1.0158x over previous
import functools

import jax
import jax.numpy as jnp
from jax import lax
from jax.experimental import pallas as pl
from jax.experimental.pallas import tpu as pltpu
from jax.experimental.pallas import tpu_sc as plsc

_DIM = 32
_CODES = 512
_BM = 4096


def _argmin_body(n_tokens, x_ref, emb_ref, idx_ref, loss_ref, tab_ref):
    i = pl.program_id(0)
    x = x_ref[...]
    e = emb_ref[...]
    rowsq = jnp.sum(x * x, axis=1, keepdims=True)
    esq = jnp.sum(e * e, axis=0, keepdims=True)
    dot = jnp.dot(x, e, preferred_element_type=jnp.float32)
    dist = (rowsq - 2.0 * dot) + esq
    minval = jnp.min(dist, axis=1, keepdims=True)
    cols = lax.broadcasted_iota(jnp.int32, (1, _CODES), 1).astype(jnp.float32)
    idxf = jnp.min(jnp.where(dist == minval, cols, float(_CODES)), axis=1)
    idx_ref[...] = idxf.astype(jnp.int32).reshape(_BM // 256, 256)
    part = jnp.dot(jnp.ones((1, _BM), jnp.float32), minval,
                   preferred_element_type=jnp.float32)

    @pl.when(i == 0)
    def _init():
        loss_ref[...] = jnp.zeros((1, 1), jnp.float32)
        tab_ref[...] = e.T

    loss_ref[...] += part

    @pl.when(i == pl.num_programs(0) - 1)
    def _scale():
        loss_ref[...] *= 1.25 / (n_tokens * _DIM)


def _tc_argmin(flat_x, embeddings, chunk, n_chunks):
    n_total = flat_x.shape[0]
    nc = n_total // n_chunks
    grid = nc // _BM
    base = chunk * grid
    return pl.pallas_call(
        functools.partial(_argmin_body, n_total),
        grid=(grid,),
        in_specs=[
            pl.BlockSpec((_BM, _DIM), lambda i: (base + i, 0)),
            pl.BlockSpec((_DIM, _CODES), lambda i: (0, 0)),
        ],
        out_specs=[
            pl.BlockSpec((_BM // 256, 256), lambda i: (i, 0)),
            pl.BlockSpec((1, 1), lambda i: (0, 0)),
            pl.BlockSpec((_CODES, _DIM), lambda i: (0, 0)),
        ],
        out_shape=[
            jax.ShapeDtypeStruct((nc // 256, 256), jnp.int32),
            jax.ShapeDtypeStruct((1, 1), jnp.float32),
            jax.ShapeDtypeStruct((_CODES, _DIM), jnp.float32),
        ],
    )(flat_x, embeddings)


_SC_CORES = 2
_SC_SUBCORES = 16


def _make_sc_gather(n_tokens):
    nw = _SC_CORES * _SC_SUBCORES
    rows_per_w = n_tokens // nw
    fire = 256
    group = 4
    gtok = group * fire
    n_groups = rows_per_w // gtok
    n_idx_rows = rows_per_w // fire
    d0 = n_tokens // gtok
    mesh = plsc.VectorSubcoreMesh(
        core_axis_name="c", subcore_axis_name="s",
        num_cores=_SC_CORES, num_subcores=_SC_SUBCORES)

    @functools.partial(
        pl.kernel,
        mesh=mesh,
        out_type=jax.ShapeDtypeStruct((d0, gtok, _DIM), jnp.float32),
        scratch_types=[
            pltpu.VMEM((n_idx_rows, fire), jnp.int32),
            pltpu.VMEM((2, gtok, _DIM), jnp.float32),
            pltpu.SemaphoreType.DMA,
            pltpu.SemaphoreType.DMA,
            pltpu.SemaphoreType.DMA,
            pltpu.SemaphoreType.DMA,
        ],
        compiler_params=pltpu.CompilerParams(use_tc_tiling_on_sc=False),
    )
    def gather_kernel(table_hbm, idx_hbm, out_hbm, idx_v, rows_v,
                      gsem0, gsem1, ssem0, ssem1):
        wid = lax.axis_index("s") * _SC_CORES + lax.axis_index("c")
        gsems = (gsem0, gsem1)
        ssems = (ssem0, ssem1)
        pltpu.sync_copy(idx_hbm.at[pl.ds(wid * n_idx_rows, n_idx_rows)], idx_v)

        def fire_gathers(g, b):
            return [pltpu.async_copy(
                table_hbm.at[idx_v.at[g * group + t]],
                rows_v.at[b, pl.ds(t * fire, fire)],
                gsems[b]) for t in range(group)]

        gh = {0: fire_gathers(0, 0)}
        sh = {}
        for g in range(n_groups):
            b = g & 1
            nb = b ^ 1
            if g + 1 < n_groups:
                if g - 1 in sh:
                    sh.pop(g - 1).wait()
                gh[g + 1] = fire_gathers(g + 1, nb)
            for c in gh.pop(g):
                c.wait()
            sh[g] = pltpu.async_copy(
                rows_v.at[b], out_hbm.at[wid * n_groups + g], ssems[b])
        for g in sorted(sh):
            sh.pop(g).wait()

    return gather_kernel


def kernel(inputs, embeddings):
    in_shape = inputs.shape
    flat_x = inputs.reshape(-1, _DIM)
    n = flat_x.shape[0]
    n_chunks = 2
    gather = _make_sc_gather(n // n_chunks)
    parts, losses, table = [], [], None
    for c in range(n_chunks):
        idx_c, loss_c, tab_c = _tc_argmin(flat_x, embeddings, c, n_chunks)
        losses.append(loss_c)
        if table is None:
            table = tab_c
        parts.append(gather(table, idx_c))
    quant = jnp.concatenate(parts, axis=0)
    loss = sum(l[0, 0] for l in losses)
    return quant.reshape(in_shape), loss
